# baseline (speedup 1.0000x reference)
import jax
import jax.numpy as jnp
from jax.experimental import pallas as pl
from jax.experimental.pallas import tpu as pltpu

TN = 2048
TM = 1024


def _pack(t, swap):
    tsq = jnp.sum(t * t, axis=0, keepdims=True)
    one = jnp.ones_like(tsq)
    if swap:
        c = jnp.concatenate([t, one, tsq], axis=0)
    else:
        c = jnp.concatenate([t * -2.0, tsq, one], axis=0)
    ch = c.astype(jnp.bfloat16).astype(jnp.float32)
    parts = [ch, ch, c - ch] if not swap else [ch, c - ch, ch]
    return jnp.concatenate(parts, axis=0).astype(jnp.bfloat16)


def _chamfer_kernel(x_ref, yt_ref, rowmin_ref, colmin_ref, b2_ref):
    i = pl.program_id(1)
    M = yt_ref.shape[2]

    @pl.when(i == 0)
    def _():
        b2_ref[...] = _pack(yt_ref[0], swap=True)

    xtb = x_ref[0]
    a2t = _pack(xtb, swap=False)

    rm128 = None
    cms = []
    for j in range(M // TM):
        d = jax.lax.dot_general(
            a2t, b2_ref[:, j * TM:(j + 1) * TM],
            (((0,), (0,)), ((), ())),
            preferred_element_type=jnp.float32)
        cms.append(jnp.min(d, axis=0))
        p = d[:, 0:128]
        for k in range(1, TM // 128):
            p = jnp.minimum(p, d[:, k * 128:(k + 1) * 128])
        rm128 = p if rm128 is None else jnp.minimum(rm128, p)
    rowmin_ref[0, 0] = jnp.min(rm128, axis=1)
    cm = jnp.concatenate(cms)

    @pl.when(i == 0)
    def _():
        colmin_ref[0, 0] = cm

    @pl.when(i != 0)
    def _():
        colmin_ref[0, 0] = jnp.minimum(colmin_ref[0, 0], cm)


def kernel(x, y):
    B, N, _ = x.shape
    M = y.shape[1]
    xt = jnp.transpose(x, (0, 2, 1))
    yt = jnp.transpose(y, (0, 2, 1))
    rowmin, colmin = pl.pallas_call(
        _chamfer_kernel,
        grid=(B, N // TN),
        in_specs=[
            pl.BlockSpec((1, 3, TN), lambda b, i: (b, 0, i)),
            pl.BlockSpec((1, 3, M), lambda b, i: (b, 0, 0)),
        ],
        out_specs=[
            pl.BlockSpec((1, 1, TN), lambda b, i: (b, 0, i)),
            pl.BlockSpec((1, 1, M), lambda b, i: (b, 0, 0)),
        ],
        out_shape=[
            jax.ShapeDtypeStruct((B, 1, N), jnp.float32),
            jax.ShapeDtypeStruct((B, 1, M), jnp.float32),
        ],
        scratch_shapes=[pltpu.VMEM((15, M), jnp.bfloat16)],
    )(xt, yt)
    return jnp.mean(rowmin) + jnp.mean(colmin)


# speedup vs baseline: 1.0123x; 1.0123x over previous
import jax
import jax.numpy as jnp
from jax.experimental import pallas as pl
from jax.experimental.pallas import tpu as pltpu

TN = 1024
TM = 1024


def _pack(t, swap):
    tsq = jnp.sum(t * t, axis=0, keepdims=True)
    one = jnp.ones_like(tsq)
    if swap:
        c = jnp.concatenate([t, one, tsq], axis=0)
    else:
        c = jnp.concatenate([t * -2.0, tsq, one], axis=0)
    ch = c.astype(jnp.bfloat16).astype(jnp.float32)
    parts = [ch, ch, c - ch] if not swap else [ch, c - ch, ch]
    return jnp.concatenate(parts, axis=0).astype(jnp.bfloat16)


def _chamfer_kernel(x_ref, yt_ref, rowmin_ref, colmin_ref, b2_ref):
    i = pl.program_id(1)
    M = yt_ref.shape[2]

    @pl.when(i == 0)
    def _():
        b2_ref[...] = _pack(yt_ref[0], swap=True)

    xtb = x_ref[0]
    a2t = _pack(xtb, swap=False)

    rm128 = None
    cms = []
    for j in range(M // TM):
        d = jax.lax.dot_general(
            a2t, b2_ref[:, j * TM:(j + 1) * TM],
            (((0,), (0,)), ((), ())),
            preferred_element_type=jnp.float32)
        cms.append(jnp.min(d, axis=0))
        p = d[:, 0:128]
        for k in range(1, TM // 128):
            p = jnp.minimum(p, d[:, k * 128:(k + 1) * 128])
        rm128 = p if rm128 is None else jnp.minimum(rm128, p)
    rowmin_ref[0, 0] = jnp.min(rm128, axis=1)
    cm = jnp.concatenate(cms)

    @pl.when(i == 0)
    def _():
        colmin_ref[0, 0] = cm

    @pl.when(i != 0)
    def _():
        colmin_ref[0, 0] = jnp.minimum(colmin_ref[0, 0], cm)


def kernel(x, y):
    B, N, _ = x.shape
    M = y.shape[1]
    xt = jnp.transpose(x, (0, 2, 1))
    yt = jnp.transpose(y, (0, 2, 1))
    rowmin, colmin = pl.pallas_call(
        _chamfer_kernel,
        grid=(B, N // TN),
        in_specs=[
            pl.BlockSpec((1, 3, TN), lambda b, i: (b, 0, i)),
            pl.BlockSpec((1, 3, M), lambda b, i: (b, 0, 0)),
        ],
        out_specs=[
            pl.BlockSpec((1, 1, TN), lambda b, i: (b, 0, i)),
            pl.BlockSpec((1, 1, M), lambda b, i: (b, 0, 0)),
        ],
        out_shape=[
            jax.ShapeDtypeStruct((B, 1, N), jnp.float32),
            jax.ShapeDtypeStruct((B, 1, M), jnp.float32),
        ],
        scratch_shapes=[pltpu.VMEM((15, M), jnp.bfloat16)],
    )(xt, yt)
    return jnp.mean(rowmin) + jnp.mean(colmin)


# speedup vs baseline: 1.4431x; 1.4256x over previous
import jax
import jax.numpy as jnp
from jax.experimental import pallas as pl
from jax.experimental.pallas import tpu as pltpu

TN = 1024
TM = 1024


def _pack(t, swap):
    tsq = jnp.sum(t * t, axis=0, keepdims=True)
    one = jnp.ones_like(tsq)
    if swap:
        c = jnp.concatenate([t, one, tsq], axis=0)
    else:
        c = jnp.concatenate([t * -2.0, tsq, one], axis=0)
    ch = c.astype(jnp.bfloat16).astype(jnp.float32)
    parts = [ch, ch, c - ch] if not swap else [ch, c - ch, ch]
    return jnp.concatenate(parts, axis=0).astype(jnp.bfloat16)


def _chamfer_kernel(x_ref, yt_ref, rowmin_ref, colmin_ref, b2_ref):
    i = pl.program_id(1)
    M = yt_ref.shape[2]

    @pl.when(i == 0)
    def _():
        b2_ref[...] = _pack(yt_ref[0], swap=True)

    xtb = x_ref[0]
    a2t = _pack(xtb, swap=False)

    rm128 = None
    cms = []
    for j in range(M // TM):
        d = jax.lax.dot_general(
            a2t, b2_ref[:, j * TM:(j + 1) * TM],
            (((0,), (0,)), ((), ())),
            preferred_element_type=jnp.float32)
        cms.append(jnp.min(d, axis=0))
        p = d[:, 0:128]
        for k in range(1, TM // 128):
            p = jnp.minimum(p, d[:, k * 128:(k + 1) * 128])
        rm128 = p if rm128 is None else jnp.minimum(rm128, p)
    rmT = jnp.swapaxes(rm128, 0, 1)
    q = rmT[0:8]
    for k in range(1, 16):
        q = jnp.minimum(q, rmT[8 * k:8 * (k + 1)])
    rowmin_ref[0, 0] = jnp.min(q, axis=0)
    cm = jnp.concatenate(cms)

    @pl.when(i == 0)
    def _():
        colmin_ref[0, 0] = cm

    @pl.when(i != 0)
    def _():
        colmin_ref[0, 0] = jnp.minimum(colmin_ref[0, 0], cm)


def kernel(x, y):
    B, N, _ = x.shape
    M = y.shape[1]
    xt = jnp.transpose(x, (0, 2, 1))
    yt = jnp.transpose(y, (0, 2, 1))
    rowmin, colmin = pl.pallas_call(
        _chamfer_kernel,
        grid=(B, N // TN),
        in_specs=[
            pl.BlockSpec((1, 3, TN), lambda b, i: (b, 0, i)),
            pl.BlockSpec((1, 3, M), lambda b, i: (b, 0, 0)),
        ],
        out_specs=[
            pl.BlockSpec((1, 1, TN), lambda b, i: (b, 0, i)),
            pl.BlockSpec((1, 1, M), lambda b, i: (b, 0, 0)),
        ],
        out_shape=[
            jax.ShapeDtypeStruct((B, 1, N), jnp.float32),
            jax.ShapeDtypeStruct((B, 1, M), jnp.float32),
        ],
        scratch_shapes=[pltpu.VMEM((15, M), jnp.bfloat16)],
    )(xt, yt)
    return jnp.mean(rowmin) + jnp.mean(colmin)


# speedup vs baseline: 1.5432x; 1.0694x over previous
import jax
import jax.numpy as jnp
from jax.experimental import pallas as pl
from jax.experimental.pallas import tpu as pltpu

TN = 2048
TM = 1024


def _pack(t, swap):
    tsq = jnp.sum(t * t, axis=0, keepdims=True)
    one = jnp.ones_like(tsq)
    if swap:
        c = jnp.concatenate([t, one, tsq], axis=0)
    else:
        c = jnp.concatenate([t * -2.0, tsq, one], axis=0)
    ch = c.astype(jnp.bfloat16).astype(jnp.float32)
    parts = [ch, ch, c - ch] if not swap else [ch, c - ch, ch]
    return jnp.concatenate(parts, axis=0).astype(jnp.bfloat16)


def _chamfer_kernel(x_ref, yt_ref, rowmin_ref, colmin_ref, b2_ref):
    i = pl.program_id(1)
    M = yt_ref.shape[2]

    @pl.when(i == 0)
    def _():
        b2_ref[...] = _pack(yt_ref[0], swap=True)

    xtb = x_ref[0]
    a2t = _pack(xtb, swap=False)

    rm128 = None
    cms = []
    for j in range(M // TM):
        d = jax.lax.dot_general(
            a2t, b2_ref[:, j * TM:(j + 1) * TM],
            (((0,), (0,)), ((), ())),
            preferred_element_type=jnp.float32)
        cms.append(jnp.min(d, axis=0))
        p = d[:, 0:128]
        for k in range(1, TM // 128):
            p = jnp.minimum(p, d[:, k * 128:(k + 1) * 128])
        rm128 = p if rm128 is None else jnp.minimum(rm128, p)
    rmT = jnp.swapaxes(rm128, 0, 1)
    q = rmT[0:8]
    for k in range(1, 16):
        q = jnp.minimum(q, rmT[8 * k:8 * (k + 1)])
    rowmin_ref[0, 0] = jnp.min(q, axis=0)
    cm = jnp.concatenate(cms)

    @pl.when(i == 0)
    def _():
        colmin_ref[0, 0] = cm

    @pl.when(i != 0)
    def _():
        colmin_ref[0, 0] = jnp.minimum(colmin_ref[0, 0], cm)


def kernel(x, y):
    B, N, _ = x.shape
    M = y.shape[1]
    xt = jnp.transpose(x, (0, 2, 1))
    yt = jnp.transpose(y, (0, 2, 1))
    rowmin, colmin = pl.pallas_call(
        _chamfer_kernel,
        grid=(B, N // TN),
        in_specs=[
            pl.BlockSpec((1, 3, TN), lambda b, i: (b, 0, i)),
            pl.BlockSpec((1, 3, M), lambda b, i: (b, 0, 0)),
        ],
        out_specs=[
            pl.BlockSpec((1, 1, TN), lambda b, i: (b, 0, i)),
            pl.BlockSpec((1, 1, M), lambda b, i: (b, 0, 0)),
        ],
        out_shape=[
            jax.ShapeDtypeStruct((B, 1, N), jnp.float32),
            jax.ShapeDtypeStruct((B, 1, M), jnp.float32),
        ],
        scratch_shapes=[pltpu.VMEM((15, M), jnp.bfloat16)],
    )(xt, yt)
    return jnp.mean(rowmin) + jnp.mean(colmin)


# speedup vs baseline: 1.6171x; 1.0479x over previous
import jax
import jax.numpy as jnp
from jax.experimental import pallas as pl
from jax.experimental.pallas import tpu as pltpu

TN = 4096
TM = 1024


def _pack(t, swap):
    tsq = jnp.sum(t * t, axis=0, keepdims=True)
    one = jnp.ones_like(tsq)
    if swap:
        c = jnp.concatenate([t, one, tsq], axis=0)
    else:
        c = jnp.concatenate([t * -2.0, tsq, one], axis=0)
    ch = c.astype(jnp.bfloat16).astype(jnp.float32)
    parts = [ch, ch, c - ch] if not swap else [ch, c - ch, ch]
    return jnp.concatenate(parts, axis=0).astype(jnp.bfloat16)


def _chamfer_kernel(x_ref, yt_ref, rowmin_ref, colmin_ref, b2_ref):
    i = pl.program_id(1)
    M = yt_ref.shape[2]

    @pl.when(i == 0)
    def _():
        b2_ref[...] = _pack(yt_ref[0], swap=True)

    xtb = x_ref[0]
    a2t = _pack(xtb, swap=False)

    rm128 = None
    cms = []
    for j in range(M // TM):
        d = jax.lax.dot_general(
            a2t, b2_ref[:, j * TM:(j + 1) * TM],
            (((0,), (0,)), ((), ())),
            preferred_element_type=jnp.float32)
        cms.append(jnp.min(d, axis=0))
        p = d[:, 0:128]
        for k in range(1, TM // 128):
            p = jnp.minimum(p, d[:, k * 128:(k + 1) * 128])
        rm128 = p if rm128 is None else jnp.minimum(rm128, p)
    rmT = jnp.swapaxes(rm128, 0, 1)
    q = rmT[0:8]
    for k in range(1, 16):
        q = jnp.minimum(q, rmT[8 * k:8 * (k + 1)])
    rowmin_ref[0, 0] = jnp.min(q, axis=0)
    cm = jnp.concatenate(cms)

    @pl.when(i == 0)
    def _():
        colmin_ref[0, 0] = cm

    @pl.when(i != 0)
    def _():
        colmin_ref[0, 0] = jnp.minimum(colmin_ref[0, 0], cm)


def kernel(x, y):
    B, N, _ = x.shape
    M = y.shape[1]
    xt = jnp.transpose(x, (0, 2, 1))
    yt = jnp.transpose(y, (0, 2, 1))
    rowmin, colmin = pl.pallas_call(
        _chamfer_kernel,
        grid=(B, N // TN),
        in_specs=[
            pl.BlockSpec((1, 3, TN), lambda b, i: (b, 0, i)),
            pl.BlockSpec((1, 3, M), lambda b, i: (b, 0, 0)),
        ],
        out_specs=[
            pl.BlockSpec((1, 1, TN), lambda b, i: (b, 0, i)),
            pl.BlockSpec((1, 1, M), lambda b, i: (b, 0, 0)),
        ],
        out_shape=[
            jax.ShapeDtypeStruct((B, 1, N), jnp.float32),
            jax.ShapeDtypeStruct((B, 1, M), jnp.float32),
        ],
        scratch_shapes=[pltpu.VMEM((15, M), jnp.bfloat16)],
    )(xt, yt)
    return jnp.mean(rowmin) + jnp.mean(colmin)
